# baseline (speedup 1.0000x reference)
import functools

import jax
import jax.numpy as jnp
import numpy as np
from jax import lax
from jax.experimental import pallas as pl
from jax.experimental.pallas import tpu as pltpu
from jax.experimental.pallas import tpu_sc as plsc

N_NODES = 10000
IN_CH = 128
HID = 64

NC, NS = 2, 16
NW = NC * NS
EB = 128
NBUF = 4
K0, K1 = 80, 80
NPAD = 10240
RPT = NPAD // NS
RB = 1024

PERM = np.concatenate(
    [np.concatenate([np.arange(g * 32, g * 32 + 32, 2),
                     np.arange(g * 32 + 1, g * 32 + 32, 2)])
     for g in range(HID // 32)])
INVPERM = np.argsort(PERM)


def _sc_degree(dst_idx, ones_v, zcol):
    maxk = max(K0, K1)
    mesh = plsc.VectorSubcoreMesh(core_axis_name="c", subcore_axis_name="s")

    @functools.partial(
        pl.kernel,
        out_type=jax.ShapeDtypeStruct((NC, NPAD), jnp.float32),
        mesh=mesh,
        scratch_types=[
            pltpu.VMEM((maxk, EB), jnp.int32),
            pltpu.VMEM((EB,), jnp.float32),
            pltpu.VMEM_SHARED((NPAD,), jnp.float32),
        ],
        compiler_params=pltpu.CompilerParams(use_tc_tiling_on_sc=False),
    )
    def k(dst_hbm, ones_hbm, zcol_hbm, out_hbm, idx_v, ones_vm, hist_sh):
        cid = lax.axis_index("c")
        sid = lax.axis_index("s")
        base = jnp.where(cid == 0, sid * K0, NS * K0 + sid * K1)
        count = jnp.where(cid == 0, K0, K1)
        pltpu.sync_copy(dst_hbm.at[pl.ds(base, maxk)], idx_v)
        pltpu.sync_copy(ones_hbm, ones_vm)
        pltpu.sync_copy(zcol_hbm, hist_sh.at[pl.ds(sid * RPT, RPT)])
        plsc.subcore_barrier()

        def body(c, carry):
            pltpu.sync_copy(ones_vm, hist_sh.at[idx_v.at[c]], add=True)
            return carry

        lax.fori_loop(0, count, body, 0)
        plsc.subcore_barrier()
        pltpu.sync_copy(hist_sh.at[pl.ds(sid * RPT, RPT)],
                        out_hbm.at[cid, pl.ds(sid * RPT, RPT)])

    return k(dst_idx, ones_v, zcol)


def _sc_aggregate(ybf, src_idx, dst_idx, zrows):
    maxk = max(K0, K1)
    mesh = plsc.VectorSubcoreMesh(core_axis_name="c", subcore_axis_name="s")

    @functools.partial(
        pl.kernel,
        out_type=jax.ShapeDtypeStruct((NC, NPAD, HID), jnp.float32),
        mesh=mesh,
        scratch_types=(
            [pltpu.VMEM((maxk, EB), jnp.int32),
             pltpu.VMEM((maxk, EB), jnp.int32)]
            + [pltpu.VMEM((EB, HID), jnp.bfloat16) for _ in range(NBUF)]
            + [pltpu.VMEM((EB, HID), jnp.float32) for _ in range(NBUF)]
            + [pltpu.VMEM_SHARED((NPAD, HID), jnp.float32)]
            + [pltpu.SemaphoreType.DMA for _ in range(2 * NBUF)]
        ),
        compiler_params=pltpu.CompilerParams(use_tc_tiling_on_sc=False,
                                             needs_layout_passes=False),
    )
    def k(y_hbm, src_hbm, dst_hbm, z_hbm, out_hbm,
          srcv, dstv, *rest):
        bbufs = rest[0:NBUF]
        fbufs = rest[NBUF:2 * NBUF]
        acc_sh = rest[2 * NBUF]
        gsems = rest[2 * NBUF + 1: 3 * NBUF + 1]
        ssems = rest[3 * NBUF + 1:]
        cid = lax.axis_index("c")
        sid = lax.axis_index("s")
        base = jnp.where(cid == 0, sid * K0, NS * K0 + sid * K1)
        count = jnp.where(cid == 0, K0, K1)
        pltpu.sync_copy(src_hbm.at[pl.ds(base, maxk)], srcv)
        pltpu.sync_copy(dst_hbm.at[pl.ds(base, maxk)], dstv)
        pltpu.sync_copy(z_hbm, acc_sh.at[pl.ds(sid * RPT, RPT)])
        plsc.subcore_barrier()

        himask = jnp.full((16,), -65536, jnp.int32)

        def widen(bf, fb):
            def crow(i, carry):
                for g in range(HID // 32):
                    iv = plsc.bitcast(bf[i, pl.ds(32 * g, 32)], jnp.int32)
                    lo = plsc.bitcast(iv << 16, jnp.float32)
                    hi = plsc.bitcast(iv & himask, jnp.float32)
                    fb[i, pl.ds(32 * g, 16)] = lo
                    fb[i, pl.ds(32 * g + 16, 16)] = hi
                return carry
            lax.fori_loop(0, EB, crow, 0)

        for b in range(NBUF):
            pltpu.async_copy(y_hbm.at[srcv.at[b]], bbufs[b], gsems[b])

        def outer(g, carry):
            for b in range(NBUF):
                c = g * NBUF + b

                @pl.when(c >= NBUF)
                def _():
                    pltpu.make_async_copy(
                        fbufs[b], acc_sh.at[dstv.at[c - NBUF]],
                        ssems[b]).wait()

                pltpu.make_async_copy(
                    y_hbm.at[srcv.at[c]], bbufs[b], gsems[b]).wait()
                widen(bbufs[b], fbufs[b])

                @pl.when(c + NBUF < count)
                def _():
                    pltpu.async_copy(
                        y_hbm.at[srcv.at[c + NBUF]], bbufs[b], gsems[b])

                pltpu.async_copy(
                    fbufs[b], acc_sh.at[dstv.at[c]], ssems[b], add=True)
            return carry

        lax.fori_loop(0, count // NBUF, outer, 0)
        for b in range(NBUF):
            pltpu.make_async_copy(
                fbufs[b], acc_sh.at[dstv.at[count - NBUF + b]],
                ssems[b]).wait()
        plsc.subcore_barrier()
        pltpu.sync_copy(acc_sh.at[pl.ds(sid * RPT, RPT)],
                        out_hbm.at[cid, pl.ds(sid * RPT, RPT)])

    return k(ybf, src_idx, dst_idx, zrows)


def _tc_prep(degs_t, x, w1, w1p):
    in_ch = x.shape[1]
    nb = NPAD // RB

    def body(deg_ref, x_ref, w_ref, wp_ref, dis_ref, y_ref, ybf_ref):
        d = deg_ref[:, 0:1] + deg_ref[:, 1:2] + 1.0
        dv = lax.rsqrt(d)
        dis_ref[...] = dv
        xv = x_ref[...]
        y_ref[...] = jnp.dot(xv, wp_ref[...],
                             preferred_element_type=jnp.float32) * dv
        ybf_ref[...] = (jnp.dot(xv, w_ref[...],
                                preferred_element_type=jnp.float32)
                        * dv).astype(jnp.bfloat16)

    return pl.pallas_call(
        body,
        grid=(nb,),
        in_specs=[
            pl.BlockSpec((RB, NC), lambda i: (i, 0)),
            pl.BlockSpec((RB, in_ch), lambda i: (i, 0)),
            pl.BlockSpec((in_ch, HID), lambda i: (0, 0)),
            pl.BlockSpec((in_ch, HID), lambda i: (0, 0)),
        ],
        out_specs=[
            pl.BlockSpec((RB, 1), lambda i: (i, 0)),
            pl.BlockSpec((RB, HID), lambda i: (i, 0)),
            pl.BlockSpec((RB, HID), lambda i: (i, 0)),
        ],
        out_shape=[
            jax.ShapeDtypeStruct((NPAD, 1), jnp.float32),
            jax.ShapeDtypeStruct((NPAD, HID), jnp.float32),
            jax.ShapeDtypeStruct((NPAD, HID), jnp.bfloat16),
        ],
        compiler_params=pltpu.CompilerParams(
            dimension_semantics=("parallel",)),
    )(degs_t, x, w1, w1p)


def _tc_mid(dis, a0, a1, y1, b1, w2a, w2b):
    nb = NPAD // RB

    def body(dis_ref, a0_ref, a1_ref, y_ref, b_ref, wa_ref, wb_ref,
             out_ref, obf_ref):
        i = pl.program_id(0)
        s = a0_ref[...] + a1_ref[...] + y_ref[...]
        h = jnp.maximum(s * dis_ref[...] + b_ref[...], 0.0)
        row = i * RB + lax.broadcasted_iota(jnp.int32, (RB, 1), 0)
        h = jnp.where(row < N_NODES, h, 0.0)
        dv = dis_ref[...]
        out_ref[...] = jnp.dot(h, wb_ref[...],
                               preferred_element_type=jnp.float32) * dv
        obf_ref[...] = (jnp.dot(h, wa_ref[...],
                                preferred_element_type=jnp.float32)
                        * dv).astype(jnp.bfloat16)

    return pl.pallas_call(
        body,
        grid=(nb,),
        in_specs=[
            pl.BlockSpec((RB, 1), lambda i: (i, 0)),
            pl.BlockSpec((RB, HID), lambda i: (i, 0)),
            pl.BlockSpec((RB, HID), lambda i: (i, 0)),
            pl.BlockSpec((RB, HID), lambda i: (i, 0)),
            pl.BlockSpec((1, HID), lambda i: (0, 0)),
            pl.BlockSpec((HID, HID), lambda i: (0, 0)),
            pl.BlockSpec((HID, HID), lambda i: (0, 0)),
        ],
        out_specs=[
            pl.BlockSpec((RB, HID), lambda i: (i, 0)),
            pl.BlockSpec((RB, HID), lambda i: (i, 0)),
        ],
        out_shape=[
            jax.ShapeDtypeStruct((NPAD, HID), jnp.float32),
            jax.ShapeDtypeStruct((NPAD, HID), jnp.bfloat16),
        ],
        compiler_params=pltpu.CompilerParams(
            dimension_semantics=("parallel",)),
    )(dis, a0, a1, y1, b1, w2a, w2b)


def _tc_final(dis, a0, a1, y2, b2):
    nb = NPAD // RB

    def body(dis_ref, a0_ref, a1_ref, y_ref, b_ref, out_ref):
        s = a0_ref[...] + a1_ref[...] + y_ref[...]
        out_ref[...] = s * dis_ref[...] + b_ref[...]

    return pl.pallas_call(
        body,
        grid=(nb,),
        in_specs=[
            pl.BlockSpec((RB, 1), lambda i: (i, 0)),
            pl.BlockSpec((RB, HID), lambda i: (i, 0)),
            pl.BlockSpec((RB, HID), lambda i: (i, 0)),
            pl.BlockSpec((RB, HID), lambda i: (i, 0)),
            pl.BlockSpec((1, HID), lambda i: (0, 0)),
        ],
        out_specs=pl.BlockSpec((RB, HID), lambda i: (i, 0)),
        out_shape=jax.ShapeDtypeStruct((NPAD, HID), jnp.float32),
        compiler_params=pltpu.CompilerParams(
            dimension_semantics=("parallel",)),
    )(dis, a0, a1, y2, b2)


def kernel(x, edge_index, W1, b1, W2, b2):
    n, in_ch = x.shape
    e = edge_index.shape[1]
    cap = NS * (K0 + K1) + max(K0, K1)
    assert NS * (K0 + K1) * EB >= e

    src = edge_index[0].astype(jnp.int32)
    dst = edge_index[1].astype(jnp.int32)
    pad = jnp.full((cap * EB - e,), n, jnp.int32)
    srcp = jnp.concatenate([src, pad]).reshape(cap, EB)
    dstp = jnp.concatenate([dst, pad]).reshape(cap, EB)

    xp = jnp.zeros((NPAD, in_ch), jnp.float32).at[:n].set(x)
    ones_v = jnp.ones((EB,), jnp.float32)
    zcol = jnp.zeros((RPT,), jnp.float32)
    zrows = jnp.zeros((RPT, HID), jnp.float32)

    w1p = W1[:, PERM]
    w2a = W2[PERM, :]
    w2b = W2[PERM][:, PERM]
    b1p = b1[PERM].reshape(1, HID)
    b2p = b2[PERM].reshape(1, HID)

    degs = _sc_degree(dstp, ones_v, zcol)
    dis, y1, y1bf = _tc_prep(degs.T, xp, W1, w1p)

    acc1 = _sc_aggregate(y1bf, srcp, dstp, zrows)
    y2, y2bf = _tc_mid(dis, acc1[0], acc1[1], y1, b1p, w2a, w2b)

    acc2 = _sc_aggregate(y2bf, srcp, dstp, zrows)
    out = _tc_final(dis, acc2[0], acc2[1], y2, b2p)
    return out[:n][:, INVPERM]


# speedup vs baseline: 1.1059x; 1.1059x over previous
import functools

import jax
import jax.numpy as jnp
from jax import lax
from jax.experimental import pallas as pl
from jax.experimental.pallas import tpu as pltpu
from jax.experimental.pallas import tpu_sc as plsc

N_NODES = 10000
IN_CH = 128
HID = 64

NC, NS = 2, 16
NW = NC * NS
EB = 128
NBUF = 5
K0, K1 = 80, 80
NPAD = 10240
RPT = NPAD // NS
RB = 1024


def _sc_degree(dst_idx, ones_v, zcol):
    maxk = max(K0, K1)
    mesh = plsc.VectorSubcoreMesh(core_axis_name="c", subcore_axis_name="s")

    @functools.partial(
        pl.kernel,
        out_type=jax.ShapeDtypeStruct((NC, NPAD), jnp.float32),
        mesh=mesh,
        scratch_types=[
            pltpu.VMEM((maxk, EB), jnp.int32),
            pltpu.VMEM((EB,), jnp.float32),
            pltpu.VMEM_SHARED((NPAD,), jnp.float32),
        ],
        compiler_params=pltpu.CompilerParams(use_tc_tiling_on_sc=False),
    )
    def k(dst_hbm, ones_hbm, zcol_hbm, out_hbm, idx_v, ones_vm, hist_sh):
        cid = lax.axis_index("c")
        sid = lax.axis_index("s")
        base = jnp.where(cid == 0, sid * K0, NS * K0 + sid * K1)
        count = jnp.where(cid == 0, K0, K1)
        pltpu.sync_copy(dst_hbm.at[pl.ds(base, maxk)], idx_v)
        pltpu.sync_copy(ones_hbm, ones_vm)
        pltpu.sync_copy(zcol_hbm, hist_sh.at[pl.ds(sid * RPT, RPT)])
        plsc.subcore_barrier()

        def body(c, carry):
            pltpu.sync_copy(ones_vm, hist_sh.at[idx_v.at[c]], add=True)
            return carry

        lax.fori_loop(0, count, body, 0)
        plsc.subcore_barrier()
        pltpu.sync_copy(hist_sh.at[pl.ds(sid * RPT, RPT)],
                        out_hbm.at[cid, pl.ds(sid * RPT, RPT)])

    return k(dst_idx, ones_v, zcol)


def _sc_aggregate(ybf, src_idx, dst_idx, zrows):
    maxk = max(K0, K1)
    mesh = plsc.VectorSubcoreMesh(core_axis_name="c", subcore_axis_name="s")

    @functools.partial(
        pl.kernel,
        out_type=jax.ShapeDtypeStruct((NC, NPAD, HID), jnp.float32),
        mesh=mesh,
        scratch_types=(
            [pltpu.VMEM((maxk, EB), jnp.int32),
             pltpu.VMEM((maxk, EB), jnp.int32)]
            + [pltpu.VMEM((EB, HID), jnp.bfloat16) for _ in range(NBUF)]
            + [pltpu.VMEM((EB, HID), jnp.float32) for _ in range(NBUF)]
            + [pltpu.VMEM_SHARED((NPAD, HID), jnp.float32)]
            + [pltpu.SemaphoreType.DMA for _ in range(2 * NBUF)]
        ),
        compiler_params=pltpu.CompilerParams(use_tc_tiling_on_sc=False,
                                             needs_layout_passes=False),
    )
    def k(y_hbm, src_hbm, dst_hbm, z_hbm, out_hbm,
          srcv, dstv, *rest):
        bbufs = rest[0:NBUF]
        fbufs = rest[NBUF:2 * NBUF]
        acc_sh = rest[2 * NBUF]
        gsems = rest[2 * NBUF + 1: 3 * NBUF + 1]
        ssems = rest[3 * NBUF + 1:]
        cid = lax.axis_index("c")
        sid = lax.axis_index("s")
        base = jnp.where(cid == 0, sid * K0, NS * K0 + sid * K1)
        count = jnp.where(cid == 0, K0, K1)
        pltpu.sync_copy(src_hbm.at[pl.ds(base, maxk)], srcv)
        pltpu.sync_copy(dst_hbm.at[pl.ds(base, maxk)], dstv)
        pltpu.sync_copy(z_hbm, acc_sh.at[pl.ds(sid * RPT, RPT)])
        plsc.subcore_barrier()

        lanes = lax.iota(jnp.int32, 16)
        himask = jnp.full((16,), -65536, jnp.int32)

        def widen(bf, fb):
            def crow(i, carry):
                rows = jnp.full((16,), i, jnp.int32)
                for g in range(HID // 32):
                    iv = plsc.bitcast(bf[i, pl.ds(32 * g, 32)], jnp.int32)
                    lo = plsc.bitcast(iv << 16, jnp.float32)
                    hi = plsc.bitcast(iv & himask, jnp.float32)
                    cols = 32 * g + 2 * lanes
                    plsc.store_scatter(fb, [rows, cols], lo)
                    plsc.store_scatter(fb, [rows, cols + 1], hi)
                return carry
            lax.fori_loop(0, EB, crow, 0)

        for b in range(NBUF):
            pltpu.async_copy(y_hbm.at[srcv.at[b]], bbufs[b], gsems[b])

        def outer(g, carry):
            for b in range(NBUF):
                c = g * NBUF + b

                @pl.when(c >= NBUF)
                def _():
                    pltpu.make_async_copy(
                        fbufs[b], acc_sh.at[dstv.at[c - NBUF]],
                        ssems[b]).wait()

                pltpu.make_async_copy(
                    y_hbm.at[srcv.at[c]], bbufs[b], gsems[b]).wait()
                widen(bbufs[b], fbufs[b])

                @pl.when(c + NBUF < count)
                def _():
                    pltpu.async_copy(
                        y_hbm.at[srcv.at[c + NBUF]], bbufs[b], gsems[b])

                pltpu.async_copy(
                    fbufs[b], acc_sh.at[dstv.at[c]], ssems[b], add=True)
            return carry

        lax.fori_loop(0, count // NBUF, outer, 0)
        for b in range(NBUF):
            pltpu.make_async_copy(
                fbufs[b], acc_sh.at[dstv.at[count - NBUF + b]],
                ssems[b]).wait()
        plsc.subcore_barrier()
        pltpu.sync_copy(acc_sh.at[pl.ds(sid * RPT, RPT)],
                        out_hbm.at[cid, pl.ds(sid * RPT, RPT)])

    return k(ybf, src_idx, dst_idx, zrows)


def _tc_prep(degs_t, x, w1):
    in_ch = x.shape[1]
    nb = NPAD // RB

    def body(deg_ref, x_ref, w_ref, dis_ref, y_ref, ybf_ref):
        d = deg_ref[:, 0:1] + deg_ref[:, 1:2] + 1.0
        dv = lax.rsqrt(d)
        dis_ref[...] = dv
        yv = jnp.dot(x_ref[...], w_ref[...],
                     preferred_element_type=jnp.float32) * dv
        y_ref[...] = yv
        ybf_ref[...] = yv.astype(jnp.bfloat16)

    return pl.pallas_call(
        body,
        grid=(nb,),
        in_specs=[
            pl.BlockSpec((RB, NC), lambda i: (i, 0)),
            pl.BlockSpec((RB, in_ch), lambda i: (i, 0)),
            pl.BlockSpec((in_ch, HID), lambda i: (0, 0)),
        ],
        out_specs=[
            pl.BlockSpec((RB, 1), lambda i: (i, 0)),
            pl.BlockSpec((RB, HID), lambda i: (i, 0)),
            pl.BlockSpec((RB, HID), lambda i: (i, 0)),
        ],
        out_shape=[
            jax.ShapeDtypeStruct((NPAD, 1), jnp.float32),
            jax.ShapeDtypeStruct((NPAD, HID), jnp.float32),
            jax.ShapeDtypeStruct((NPAD, HID), jnp.bfloat16),
        ],
        compiler_params=pltpu.CompilerParams(
            dimension_semantics=("parallel",)),
    )(degs_t, x, w1)


def _tc_mid(dis, a0, a1, y1, b1, w2):
    nb = NPAD // RB

    def body(dis_ref, a0_ref, a1_ref, y_ref, b_ref, w_ref, out_ref, obf_ref):
        i = pl.program_id(0)
        s = a0_ref[...] + a1_ref[...] + y_ref[...]
        h = jnp.maximum(s * dis_ref[...] + b_ref[...], 0.0)
        row = i * RB + lax.broadcasted_iota(jnp.int32, (RB, 1), 0)
        h = jnp.where(row < N_NODES, h, 0.0)
        yv = jnp.dot(h, w_ref[...],
                     preferred_element_type=jnp.float32) * dis_ref[...]
        out_ref[...] = yv
        obf_ref[...] = yv.astype(jnp.bfloat16)

    return pl.pallas_call(
        body,
        grid=(nb,),
        in_specs=[
            pl.BlockSpec((RB, 1), lambda i: (i, 0)),
            pl.BlockSpec((RB, HID), lambda i: (i, 0)),
            pl.BlockSpec((RB, HID), lambda i: (i, 0)),
            pl.BlockSpec((RB, HID), lambda i: (i, 0)),
            pl.BlockSpec((1, HID), lambda i: (0, 0)),
            pl.BlockSpec((HID, HID), lambda i: (0, 0)),
        ],
        out_specs=[
            pl.BlockSpec((RB, HID), lambda i: (i, 0)),
            pl.BlockSpec((RB, HID), lambda i: (i, 0)),
        ],
        out_shape=[
            jax.ShapeDtypeStruct((NPAD, HID), jnp.float32),
            jax.ShapeDtypeStruct((NPAD, HID), jnp.bfloat16),
        ],
        compiler_params=pltpu.CompilerParams(
            dimension_semantics=("parallel",)),
    )(dis, a0, a1, y1, b1, w2)


def _tc_final(dis, a0, a1, y2, b2):
    nb = NPAD // RB

    def body(dis_ref, a0_ref, a1_ref, y_ref, b_ref, out_ref):
        s = a0_ref[...] + a1_ref[...] + y_ref[...]
        out_ref[...] = s * dis_ref[...] + b_ref[...]

    return pl.pallas_call(
        body,
        grid=(nb,),
        in_specs=[
            pl.BlockSpec((RB, 1), lambda i: (i, 0)),
            pl.BlockSpec((RB, HID), lambda i: (i, 0)),
            pl.BlockSpec((RB, HID), lambda i: (i, 0)),
            pl.BlockSpec((RB, HID), lambda i: (i, 0)),
            pl.BlockSpec((1, HID), lambda i: (0, 0)),
        ],
        out_specs=pl.BlockSpec((RB, HID), lambda i: (i, 0)),
        out_shape=jax.ShapeDtypeStruct((NPAD, HID), jnp.float32),
        compiler_params=pltpu.CompilerParams(
            dimension_semantics=("parallel",)),
    )(dis, a0, a1, y2, b2)


def kernel(x, edge_index, W1, b1, W2, b2):
    n, in_ch = x.shape
    e = edge_index.shape[1]
    cap = NS * (K0 + K1) + max(K0, K1)
    assert NS * (K0 + K1) * EB >= e

    src = edge_index[0].astype(jnp.int32)
    dst = edge_index[1].astype(jnp.int32)
    pad = jnp.full((cap * EB - e,), n, jnp.int32)
    srcp = jnp.concatenate([src, pad]).reshape(cap, EB)
    dstp = jnp.concatenate([dst, pad]).reshape(cap, EB)

    xp = jnp.zeros((NPAD, in_ch), jnp.float32).at[:n].set(x)
    ones_v = jnp.ones((EB,), jnp.float32)
    zcol = jnp.zeros((RPT,), jnp.float32)
    zrows = jnp.zeros((RPT, HID), jnp.float32)

    degs = _sc_degree(dstp, ones_v, zcol)
    dis, y1, y1bf = _tc_prep(degs.T, xp, W1)

    acc1 = _sc_aggregate(y1bf, srcp, dstp, zrows)
    y2, y2bf = _tc_mid(dis, acc1[0], acc1[1], y1, b1.reshape(1, HID), W2)

    acc2 = _sc_aggregate(y2bf, srcp, dstp, zrows)
    out = _tc_final(dis, acc2[0], acc2[1], y2, b2.reshape(1, HID))
    return out[:n]


# speedup vs baseline: 1.1712x; 1.0591x over previous
import functools

import jax
import jax.numpy as jnp
from jax import lax
from jax.experimental import pallas as pl
from jax.experimental.pallas import tpu as pltpu
from jax.experimental.pallas import tpu_sc as plsc

N_NODES = 10000
IN_CH = 128
HID = 64

NC, NS = 2, 16
NW = NC * NS
EB = 128
NBUF = 4
YSROWS = 10016
K0, K1 = 80, 80
NPAD = 10240
RPT = NPAD // NS
RB = 1024


def _sc_degree(dst_idx, ones_v, zcol):
    maxk = max(K0, K1)
    mesh = plsc.VectorSubcoreMesh(core_axis_name="c", subcore_axis_name="s")

    @functools.partial(
        pl.kernel,
        out_type=jax.ShapeDtypeStruct((NC, NPAD), jnp.float32),
        mesh=mesh,
        scratch_types=[
            pltpu.VMEM((maxk, EB), jnp.int32),
            pltpu.VMEM((EB,), jnp.float32),
            pltpu.VMEM_SHARED((NPAD,), jnp.float32),
        ],
        compiler_params=pltpu.CompilerParams(use_tc_tiling_on_sc=False),
    )
    def k(dst_hbm, ones_hbm, zcol_hbm, out_hbm, idx_v, ones_vm, hist_sh):
        cid = lax.axis_index("c")
        sid = lax.axis_index("s")
        base = jnp.where(cid == 0, sid * K0, NS * K0 + sid * K1)
        count = jnp.where(cid == 0, K0, K1)
        pltpu.sync_copy(dst_hbm.at[pl.ds(base, maxk)], idx_v)
        pltpu.sync_copy(ones_hbm, ones_vm)
        pltpu.sync_copy(zcol_hbm, hist_sh.at[pl.ds(sid * RPT, RPT)])
        plsc.subcore_barrier()

        def body(c, carry):
            pltpu.sync_copy(ones_vm, hist_sh.at[idx_v.at[c]], add=True)
            return carry

        lax.fori_loop(0, count, body, 0)
        plsc.subcore_barrier()
        pltpu.sync_copy(hist_sh.at[pl.ds(sid * RPT, RPT)],
                        out_hbm.at[cid, pl.ds(sid * RPT, RPT)])

    return k(dst_idx, ones_v, zcol)


def _sc_aggregate(ybf, src_idx, dst_idx, zrows):
    maxk = max(K0, K1)
    mesh = plsc.VectorSubcoreMesh(core_axis_name="c", subcore_axis_name="s")

    @functools.partial(
        pl.kernel,
        out_type=jax.ShapeDtypeStruct((NC, NPAD, HID), jnp.float32),
        mesh=mesh,
        scratch_types=(
            [pltpu.VMEM((maxk, EB), jnp.int32),
             pltpu.VMEM((maxk, EB), jnp.int32)]
            + [pltpu.VMEM((EB, HID), jnp.bfloat16) for _ in range(NBUF)]
            + [pltpu.VMEM((EB, HID), jnp.float32) for _ in range(NBUF)]
            + [pltpu.VMEM_SHARED((NPAD, HID), jnp.float32),
               pltpu.VMEM_SHARED((YSROWS, HID), jnp.bfloat16)]
            + [pltpu.SemaphoreType.DMA for _ in range(2 * NBUF)]
        ),
        compiler_params=pltpu.CompilerParams(use_tc_tiling_on_sc=False,
                                             needs_layout_passes=False),
    )
    def k(y_hbm, src_hbm, dst_hbm, z_hbm, out_hbm,
          srcv, dstv, *rest):
        bbufs = rest[0:NBUF]
        fbufs = rest[NBUF:2 * NBUF]
        acc_sh = rest[2 * NBUF]
        y_sh = rest[2 * NBUF + 1]
        gsems = rest[2 * NBUF + 2: 3 * NBUF + 2]
        ssems = rest[3 * NBUF + 2:]
        cid = lax.axis_index("c")
        sid = lax.axis_index("s")
        base = jnp.where(cid == 0, sid * K0, NS * K0 + sid * K1)
        count = jnp.where(cid == 0, K0, K1)
        pltpu.sync_copy(src_hbm.at[pl.ds(base, maxk)], srcv)
        pltpu.sync_copy(dst_hbm.at[pl.ds(base, maxk)], dstv)
        pltpu.sync_copy(z_hbm, acc_sh.at[pl.ds(sid * RPT, RPT)])
        yr = YSROWS // NS
        pltpu.sync_copy(y_hbm.at[pl.ds(sid * yr, yr)],
                        y_sh.at[pl.ds(sid * yr, yr)])
        plsc.subcore_barrier()

        lanes = lax.iota(jnp.int32, 16)
        himask = jnp.full((16,), -65536, jnp.int32)

        def widen(bf, fb):
            def crow(i, carry):
                rows = jnp.full((16,), i, jnp.int32)
                for g in range(HID // 32):
                    iv = plsc.bitcast(bf[i, pl.ds(32 * g, 32)], jnp.int32)
                    lo = plsc.bitcast(iv << 16, jnp.float32)
                    hi = plsc.bitcast(iv & himask, jnp.float32)
                    cols = 32 * g + 2 * lanes
                    plsc.store_scatter(fb, [rows, cols], lo)
                    plsc.store_scatter(fb, [rows, cols + 1], hi)
                return carry
            lax.fori_loop(0, EB, crow, 0)

        for b in range(NBUF):
            pltpu.async_copy(y_sh.at[srcv.at[b]], bbufs[b], gsems[b])

        def outer(g, carry):
            for b in range(NBUF):
                c = g * NBUF + b

                @pl.when(c >= NBUF)
                def _():
                    pltpu.make_async_copy(
                        fbufs[b], acc_sh.at[dstv.at[c - NBUF]],
                        ssems[b]).wait()

                pltpu.make_async_copy(
                    y_sh.at[srcv.at[c]], bbufs[b], gsems[b]).wait()
                widen(bbufs[b], fbufs[b])

                @pl.when(c + NBUF < count)
                def _():
                    pltpu.async_copy(
                        y_sh.at[srcv.at[c + NBUF]], bbufs[b], gsems[b])

                pltpu.async_copy(
                    fbufs[b], acc_sh.at[dstv.at[c]], ssems[b], add=True)
            return carry

        lax.fori_loop(0, count // NBUF, outer, 0)
        for b in range(NBUF):
            pltpu.make_async_copy(
                fbufs[b], acc_sh.at[dstv.at[count - NBUF + b]],
                ssems[b]).wait()
        plsc.subcore_barrier()
        pltpu.sync_copy(acc_sh.at[pl.ds(sid * RPT, RPT)],
                        out_hbm.at[cid, pl.ds(sid * RPT, RPT)])

    return k(ybf, src_idx, dst_idx, zrows)


def _tc_prep(degs_t, x, w1):
    in_ch = x.shape[1]
    nb = NPAD // RB

    def body(deg_ref, x_ref, w_ref, dis_ref, y_ref, ybf_ref):
        d = deg_ref[:, 0:1] + deg_ref[:, 1:2] + 1.0
        dv = lax.rsqrt(d)
        dis_ref[...] = dv
        yv = jnp.dot(x_ref[...], w_ref[...],
                     preferred_element_type=jnp.float32) * dv
        y_ref[...] = yv
        ybf_ref[...] = yv.astype(jnp.bfloat16)

    return pl.pallas_call(
        body,
        grid=(nb,),
        in_specs=[
            pl.BlockSpec((RB, NC), lambda i: (i, 0)),
            pl.BlockSpec((RB, in_ch), lambda i: (i, 0)),
            pl.BlockSpec((in_ch, HID), lambda i: (0, 0)),
        ],
        out_specs=[
            pl.BlockSpec((RB, 1), lambda i: (i, 0)),
            pl.BlockSpec((RB, HID), lambda i: (i, 0)),
            pl.BlockSpec((RB, HID), lambda i: (i, 0)),
        ],
        out_shape=[
            jax.ShapeDtypeStruct((NPAD, 1), jnp.float32),
            jax.ShapeDtypeStruct((NPAD, HID), jnp.float32),
            jax.ShapeDtypeStruct((NPAD, HID), jnp.bfloat16),
        ],
        compiler_params=pltpu.CompilerParams(
            dimension_semantics=("parallel",)),
    )(degs_t, x, w1)


def _tc_mid(dis, a0, a1, y1, b1, w2):
    nb = NPAD // RB

    def body(dis_ref, a0_ref, a1_ref, y_ref, b_ref, w_ref, out_ref, obf_ref):
        i = pl.program_id(0)
        s = a0_ref[...] + a1_ref[...] + y_ref[...]
        h = jnp.maximum(s * dis_ref[...] + b_ref[...], 0.0)
        row = i * RB + lax.broadcasted_iota(jnp.int32, (RB, 1), 0)
        h = jnp.where(row < N_NODES, h, 0.0)
        yv = jnp.dot(h, w_ref[...],
                     preferred_element_type=jnp.float32) * dis_ref[...]
        out_ref[...] = yv
        obf_ref[...] = yv.astype(jnp.bfloat16)

    return pl.pallas_call(
        body,
        grid=(nb,),
        in_specs=[
            pl.BlockSpec((RB, 1), lambda i: (i, 0)),
            pl.BlockSpec((RB, HID), lambda i: (i, 0)),
            pl.BlockSpec((RB, HID), lambda i: (i, 0)),
            pl.BlockSpec((RB, HID), lambda i: (i, 0)),
            pl.BlockSpec((1, HID), lambda i: (0, 0)),
            pl.BlockSpec((HID, HID), lambda i: (0, 0)),
        ],
        out_specs=[
            pl.BlockSpec((RB, HID), lambda i: (i, 0)),
            pl.BlockSpec((RB, HID), lambda i: (i, 0)),
        ],
        out_shape=[
            jax.ShapeDtypeStruct((NPAD, HID), jnp.float32),
            jax.ShapeDtypeStruct((NPAD, HID), jnp.bfloat16),
        ],
        compiler_params=pltpu.CompilerParams(
            dimension_semantics=("parallel",)),
    )(dis, a0, a1, y1, b1, w2)


def _tc_final(dis, a0, a1, y2, b2):
    nb = NPAD // RB

    def body(dis_ref, a0_ref, a1_ref, y_ref, b_ref, out_ref):
        s = a0_ref[...] + a1_ref[...] + y_ref[...]
        out_ref[...] = s * dis_ref[...] + b_ref[...]

    return pl.pallas_call(
        body,
        grid=(nb,),
        in_specs=[
            pl.BlockSpec((RB, 1), lambda i: (i, 0)),
            pl.BlockSpec((RB, HID), lambda i: (i, 0)),
            pl.BlockSpec((RB, HID), lambda i: (i, 0)),
            pl.BlockSpec((RB, HID), lambda i: (i, 0)),
            pl.BlockSpec((1, HID), lambda i: (0, 0)),
        ],
        out_specs=pl.BlockSpec((RB, HID), lambda i: (i, 0)),
        out_shape=jax.ShapeDtypeStruct((NPAD, HID), jnp.float32),
        compiler_params=pltpu.CompilerParams(
            dimension_semantics=("parallel",)),
    )(dis, a0, a1, y2, b2)


def kernel(x, edge_index, W1, b1, W2, b2):
    n, in_ch = x.shape
    e = edge_index.shape[1]
    cap = NS * (K0 + K1) + max(K0, K1)
    assert NS * (K0 + K1) * EB >= e

    src = edge_index[0].astype(jnp.int32)
    dst = edge_index[1].astype(jnp.int32)
    pad = jnp.full((cap * EB - e,), n, jnp.int32)
    srcp = jnp.concatenate([src, pad]).reshape(cap, EB)
    dstp = jnp.concatenate([dst, pad]).reshape(cap, EB)

    xp = jnp.zeros((NPAD, in_ch), jnp.float32).at[:n].set(x)
    ones_v = jnp.ones((EB,), jnp.float32)
    zcol = jnp.zeros((RPT,), jnp.float32)
    zrows = jnp.zeros((RPT, HID), jnp.float32)

    degs = _sc_degree(dstp, ones_v, zcol)
    dis, y1, y1bf = _tc_prep(degs.T, xp, W1)

    acc1 = _sc_aggregate(y1bf, srcp, dstp, zrows)
    y2, y2bf = _tc_mid(dis, acc1[0], acc1[1], y1, b1.reshape(1, HID), W2)

    acc2 = _sc_aggregate(y2bf, srcp, dstp, zrows)
    out = _tc_final(dis, acc2[0], acc2[1], y2, b2.reshape(1, HID))
    return out[:n]


# speedup vs baseline: 1.1714x; 1.0002x over previous
import functools

import jax
import jax.numpy as jnp
from jax import lax
from jax.experimental import pallas as pl
from jax.experimental.pallas import tpu as pltpu
from jax.experimental.pallas import tpu_sc as plsc

N_NODES = 10000
IN_CH = 128
HID = 64

NC, NS = 2, 16
NW = NC * NS
EB = 128
NBUF = 4
YSROWS = 10016
K0, K1 = 80, 80
NPAD = 10240
RPT = NPAD // NS
RB = 1024


def _sc_degree(dst_idx, ones_v, zcol):
    maxk = max(K0, K1)
    mesh = plsc.VectorSubcoreMesh(core_axis_name="c", subcore_axis_name="s")

    @functools.partial(
        pl.kernel,
        out_type=jax.ShapeDtypeStruct((NC, NPAD), jnp.float32),
        mesh=mesh,
        scratch_types=[
            pltpu.VMEM((maxk, EB), jnp.int32),
            pltpu.VMEM((EB,), jnp.float32),
            pltpu.VMEM_SHARED((NPAD,), jnp.float32),
        ],
        compiler_params=pltpu.CompilerParams(use_tc_tiling_on_sc=False),
    )
    def k(dst_hbm, ones_hbm, zcol_hbm, out_hbm, idx_v, ones_vm, hist_sh):
        cid = lax.axis_index("c")
        sid = lax.axis_index("s")
        base = jnp.where(cid == 0, sid * K0, NS * K0 + sid * K1)
        count = jnp.where(cid == 0, K0, K1)
        pltpu.sync_copy(dst_hbm.at[pl.ds(base, maxk)], idx_v)
        pltpu.sync_copy(ones_hbm, ones_vm)
        pltpu.sync_copy(zcol_hbm, hist_sh.at[pl.ds(sid * RPT, RPT)])
        plsc.subcore_barrier()

        def body(c, carry):
            pltpu.sync_copy(ones_vm, hist_sh.at[idx_v.at[c]], add=True)
            return carry

        lax.fori_loop(0, count, body, 0)
        plsc.subcore_barrier()
        pltpu.sync_copy(hist_sh.at[pl.ds(sid * RPT, RPT)],
                        out_hbm.at[cid, pl.ds(sid * RPT, RPT)])

    return k(dst_idx, ones_v, zcol)


def _sc_aggregate(ybf, src_idx, dst_idx, zrows):
    maxk = max(K0, K1)
    mesh = plsc.VectorSubcoreMesh(core_axis_name="c", subcore_axis_name="s")

    @functools.partial(
        pl.kernel,
        out_type=jax.ShapeDtypeStruct((NC, NPAD, HID), jnp.float32),
        mesh=mesh,
        scratch_types=(
            [pltpu.VMEM((maxk, EB), jnp.int32),
             pltpu.VMEM((maxk, EB), jnp.int32)]
            + [pltpu.VMEM((EB, HID), jnp.bfloat16) for _ in range(NBUF)]
            + [pltpu.VMEM((EB, HID), jnp.float32) for _ in range(NBUF)]
            + [pltpu.VMEM_SHARED((NPAD, HID), jnp.float32),
               pltpu.VMEM_SHARED((YSROWS, HID), jnp.bfloat16)]
            + [pltpu.SemaphoreType.DMA for _ in range(2 * NBUF)]
        ),
        compiler_params=pltpu.CompilerParams(use_tc_tiling_on_sc=False,
                                             needs_layout_passes=False),
    )
    def k(y_hbm, src_hbm, dst_hbm, z_hbm, out_hbm,
          srcv, dstv, *rest):
        bbufs = rest[0:NBUF]
        fbufs = rest[NBUF:2 * NBUF]
        acc_sh = rest[2 * NBUF]
        y_sh = rest[2 * NBUF + 1]
        gsems = rest[2 * NBUF + 2: 3 * NBUF + 2]
        ssems = rest[3 * NBUF + 2:]
        cid = lax.axis_index("c")
        sid = lax.axis_index("s")
        base = jnp.where(cid == 0, sid * K0, NS * K0 + sid * K1)
        count = jnp.where(cid == 0, K0, K1)
        pltpu.sync_copy(src_hbm.at[pl.ds(base, maxk)], srcv)
        pltpu.sync_copy(dst_hbm.at[pl.ds(base, maxk)], dstv)
        pltpu.sync_copy(z_hbm, acc_sh.at[pl.ds(sid * RPT, RPT)])
        yr = YSROWS // NS
        pltpu.sync_copy(y_hbm.at[pl.ds(sid * yr, yr)],
                        y_sh.at[pl.ds(sid * yr, yr)])
        plsc.subcore_barrier()

        lanes = lax.iota(jnp.int32, 16)
        himask = jnp.full((16,), -65536, jnp.int32)

        def widen(bf, fb):
            def crow(i, carry):
                rows = jnp.full((16,), i, jnp.int32)
                for g in range(HID // 32):
                    iv = plsc.bitcast(bf[i, pl.ds(32 * g, 32)], jnp.int32)
                    lo = plsc.bitcast(iv << 16, jnp.float32)
                    hi = plsc.bitcast(iv & himask, jnp.float32)
                    cols = 32 * g + 2 * lanes
                    plsc.store_scatter(fb, [rows, cols], lo)
                    plsc.store_scatter(fb, [rows, cols + 1], hi)
                return carry
            lax.fori_loop(0, EB, crow, 0)

        for b in range(NBUF):
            pltpu.async_copy(y_sh.at[srcv.at[b]], bbufs[b], gsems[b])

        def outer(g, carry):
            for b in range(NBUF):
                c = g * NBUF + b

                @pl.when(c >= NBUF)
                def _():
                    pltpu.make_async_copy(
                        fbufs[b], acc_sh.at[dstv.at[c - NBUF]],
                        ssems[b]).wait()

                pltpu.make_async_copy(
                    y_sh.at[srcv.at[c]], bbufs[b], gsems[b]).wait()
                widen(bbufs[b], fbufs[b])

                @pl.when(c + NBUF < count)
                def _():
                    pltpu.async_copy(
                        y_sh.at[srcv.at[c + NBUF]], bbufs[b], gsems[b])

                pltpu.async_copy(
                    fbufs[b], acc_sh.at[dstv.at[c]], ssems[b], add=True)
            return carry

        lax.fori_loop(0, count // NBUF, outer, 0)
        for b in range(NBUF):
            pltpu.make_async_copy(
                fbufs[b], acc_sh.at[dstv.at[count - NBUF + b]],
                ssems[b]).wait()
        plsc.subcore_barrier()
        pltpu.sync_copy(acc_sh.at[pl.ds(sid * RPT, RPT)],
                        out_hbm.at[cid, pl.ds(sid * RPT, RPT)])

    return k(ybf, src_idx, dst_idx, zrows)


def _tc_prep(degs_t, x, w1):
    in_ch = x.shape[1]
    nb = NPAD // RB

    def body(deg_ref, x_ref, w_ref, dis_ref, y_ref, ybf_ref):
        d = deg_ref[:, 0:1] + deg_ref[:, 1:2] + 1.0
        dv = lax.rsqrt(d)
        dis_ref[...] = dv
        yv = jnp.dot(x_ref[...], w_ref[...],
                     preferred_element_type=jnp.float32) * dv
        y_ref[...] = yv
        ybf_ref[...] = yv.astype(jnp.bfloat16)

    return pl.pallas_call(
        body,
        grid=(nb,),
        in_specs=[
            pl.BlockSpec((RB, NC), lambda i: (i, 0)),
            pl.BlockSpec((RB, in_ch), lambda i: (i, 0)),
            pl.BlockSpec((in_ch, HID), lambda i: (0, 0)),
        ],
        out_specs=[
            pl.BlockSpec((RB, 1), lambda i: (i, 0)),
            pl.BlockSpec((RB, HID), lambda i: (i, 0)),
            pl.BlockSpec((RB, HID), lambda i: (i, 0)),
        ],
        out_shape=[
            jax.ShapeDtypeStruct((NPAD, 1), jnp.float32),
            jax.ShapeDtypeStruct((NPAD, HID), jnp.float32),
            jax.ShapeDtypeStruct((NPAD, HID), jnp.bfloat16),
        ],
        compiler_params=pltpu.CompilerParams(
            dimension_semantics=("parallel",)),
    )(degs_t, x, w1)


def _tc_mid(dis, a0, a1, y1, b1, w2):
    nb = NPAD // RB

    def body(dis_ref, a0_ref, a1_ref, y_ref, b_ref, w_ref, out_ref, obf_ref):
        i = pl.program_id(0)
        s = a0_ref[...] + a1_ref[...] + y_ref[...]
        h = jnp.maximum(s * dis_ref[...] + b_ref[...], 0.0)
        row = i * RB + lax.broadcasted_iota(jnp.int32, (RB, 1), 0)
        h = jnp.where(row < N_NODES, h, 0.0)
        yv = jnp.dot(h, w_ref[...],
                     preferred_element_type=jnp.float32) * dis_ref[...]
        out_ref[...] = yv
        obf_ref[...] = yv.astype(jnp.bfloat16)

    return pl.pallas_call(
        body,
        grid=(nb,),
        in_specs=[
            pl.BlockSpec((RB, 1), lambda i: (i, 0)),
            pl.BlockSpec((RB, HID), lambda i: (i, 0)),
            pl.BlockSpec((RB, HID), lambda i: (i, 0)),
            pl.BlockSpec((RB, HID), lambda i: (i, 0)),
            pl.BlockSpec((1, HID), lambda i: (0, 0)),
            pl.BlockSpec((HID, HID), lambda i: (0, 0)),
        ],
        out_specs=[
            pl.BlockSpec((RB, HID), lambda i: (i, 0)),
            pl.BlockSpec((RB, HID), lambda i: (i, 0)),
        ],
        out_shape=[
            jax.ShapeDtypeStruct((NPAD, HID), jnp.float32),
            jax.ShapeDtypeStruct((NPAD, HID), jnp.bfloat16),
        ],
        compiler_params=pltpu.CompilerParams(
            dimension_semantics=("parallel",)),
    )(dis, a0, a1, y1, b1, w2)


def _tc_final(dis, a0, a1, y2, b2):
    nb = NPAD // RB

    def body(dis_ref, a0_ref, a1_ref, y_ref, b_ref, out_ref):
        s = a0_ref[...] + a1_ref[...] + y_ref[...]
        out_ref[...] = s * dis_ref[...] + b_ref[...]

    return pl.pallas_call(
        body,
        grid=(nb,),
        in_specs=[
            pl.BlockSpec((RB, 1), lambda i: (i, 0)),
            pl.BlockSpec((RB, HID), lambda i: (i, 0)),
            pl.BlockSpec((RB, HID), lambda i: (i, 0)),
            pl.BlockSpec((RB, HID), lambda i: (i, 0)),
            pl.BlockSpec((1, HID), lambda i: (0, 0)),
        ],
        out_specs=pl.BlockSpec((RB, HID), lambda i: (i, 0)),
        out_shape=jax.ShapeDtypeStruct((N_NODES, HID), jnp.float32),
        compiler_params=pltpu.CompilerParams(
            dimension_semantics=("parallel",)),
    )(dis, a0, a1, y2, b2)


def kernel(x, edge_index, W1, b1, W2, b2):
    n, in_ch = x.shape
    e = edge_index.shape[1]
    cap = NS * (K0 + K1) + max(K0, K1)
    assert NS * (K0 + K1) * EB >= e

    src = edge_index[0].astype(jnp.int32)
    dst = edge_index[1].astype(jnp.int32)
    pad = jnp.full((cap * EB - e,), n, jnp.int32)
    srcp = jnp.concatenate([src, pad]).reshape(cap, EB)
    dstp = jnp.concatenate([dst, pad]).reshape(cap, EB)

    xp = jnp.zeros((NPAD, in_ch), jnp.float32).at[:n].set(x)
    ones_v = jnp.ones((EB,), jnp.float32)
    zcol = jnp.zeros((RPT,), jnp.float32)
    zrows = jnp.zeros((RPT, HID), jnp.float32)

    degs = _sc_degree(dstp, ones_v, zcol)
    dis, y1, y1bf = _tc_prep(degs.T, xp, W1)

    acc1 = _sc_aggregate(y1bf, srcp, dstp, zrows)
    y2, y2bf = _tc_mid(dis, acc1[0], acc1[1], y1, b1.reshape(1, HID), W2)

    acc2 = _sc_aggregate(y2bf, srcp, dstp, zrows)
    return _tc_final(dis, acc2[0], acc2[1], y2, b2.reshape(1, HID))
